# baseline (speedup 1.0000x reference)
import functools

import jax
import jax.numpy as jnp
from jax import lax
from jax.experimental import pallas as pl
from jax.experimental.pallas import tpu as pltpu
from jax.experimental.pallas import tpu_sc as plsc

DIM = 128
_NC, _NS = 2, 16
_NW = _NC * _NS
_TAIL = 8


def out_slice(out, r, c0, n):
    return out.at[r, 0, pl.ds(c0, n)]


@functools.lru_cache(maxsize=None)
def _build_sc_gather(b: int, s: int):
    n_rows = b * s
    assert n_rows % (8 * _NW) == 0
    b_per_w = n_rows // _NW
    w_per_row = s // b_per_w
    mesh = plsc.VectorSubcoreMesh(core_axis_name="c", subcore_axis_name="s")

    @functools.partial(
        pl.kernel,
        mesh=mesh,
        out_type=[
            jax.ShapeDtypeStruct((b, 1, s, DIM), jnp.float32),
            jax.ShapeDtypeStruct((b, 1, s, DIM), jnp.float32),
        ],
        scratch_types=[
            pltpu.VMEM((b_per_w,), jnp.int32),
            pltpu.VMEM((b_per_w, DIM), jnp.float32),
            pltpu.VMEM((b_per_w - _TAIL, DIM), jnp.float32),
            pltpu.SemaphoreType.DMA,
            pltpu.SemaphoreType.DMA,
            pltpu.SemaphoreType.DMA,
        ],
        compiler_params=pltpu.CompilerParams(
            disable_bounds_checks=True,
            disable_semaphore_checks=True,
            skip_device_barrier=True,
        ),
    )
    def sc_gather(pos_hbm, cos_hbm, sin_hbm, cos_out, sin_out,
                  idx_v, cos_v, sin_v, sem_c, sem_s, sem_w):
        wid = lax.axis_index("s") * _NC + lax.axis_index("c")
        r = wid // w_per_row
        c0 = (wid % w_per_row) * b_per_w
        head = b_per_w - _TAIL
        pltpu.sync_copy(pos_hbm.at[r, pl.ds(c0, b_per_w)], idx_v)
        g_c = pltpu.async_copy(cos_hbm.at[idx_v], cos_v, sem_c)
        g_s = pltpu.async_copy(sin_hbm.at[idx_v.at[pl.ds(0, head)]],
                               sin_v, sem_s)
        g_c.wait()
        w_c = pltpu.async_copy(cos_v, out_slice(cos_out, r, c0, b_per_w),
                               sem_w)
        g_s.wait()
        w_s = pltpu.async_copy(sin_v, out_slice(sin_out, r, c0, head), sem_w)
        w_c.wait()
        g_t = pltpu.async_copy(sin_hbm.at[idx_v.at[pl.ds(head, _TAIL)]],
                               cos_v.at[pl.ds(0, _TAIL)], sem_s)
        g_t.wait()
        w_t = pltpu.async_copy(cos_v.at[pl.ds(0, _TAIL)],
                               out_slice(sin_out, r, c0 + head, _TAIL),
                               sem_w)
        w_s.wait()
        w_t.wait()

    return sc_gather


def kernel(x, position_ids, cos_cached, sin_cached):
    b, s = position_ids.shape
    cos4, sin4 = _build_sc_gather(b, s)(
        position_ids.astype(jnp.int32),
        cos_cached.astype(jnp.float32), sin_cached.astype(jnp.float32))
    return cos4.astype(x.dtype), sin4.astype(x.dtype)


# speedup vs baseline: 1.0045x; 1.0045x over previous
import functools

import jax
import jax.numpy as jnp
from jax import lax
from jax.experimental import pallas as pl
from jax.experimental.pallas import tpu as pltpu
from jax.experimental.pallas import tpu_sc as plsc

DIM = 128
_NC, _NS = 2, 16
_NW = _NC * _NS
_TAIL = 8


def out_slice(out, r, c0, n):
    return out.at[r, 0, pl.ds(c0, n)]


@functools.lru_cache(maxsize=None)
def _build_sc_gather(b: int, s: int):
    n_rows = b * s
    assert n_rows % (8 * _NW) == 0
    b_per_w = n_rows // _NW
    w_per_row = s // b_per_w
    mesh = plsc.VectorSubcoreMesh(core_axis_name="c", subcore_axis_name="s")

    @functools.partial(
        pl.kernel,
        mesh=mesh,
        out_type=[
            jax.ShapeDtypeStruct((b, 1, s, DIM), jnp.float32),
            jax.ShapeDtypeStruct((b, 1, s, DIM), jnp.float32),
        ],
        scratch_types=[
            pltpu.VMEM((b_per_w,), jnp.int32),
            pltpu.VMEM((b_per_w, DIM), jnp.float32),
            pltpu.VMEM((b_per_w - _TAIL, DIM), jnp.float32),
            pltpu.SemaphoreType.DMA,
            pltpu.SemaphoreType.DMA,
            pltpu.SemaphoreType.DMA,
        ],
    )
    def sc_gather(pos_hbm, cos_hbm, sin_hbm, cos_out, sin_out,
                  idx_v, cos_v, sin_v, sem_c, sem_s, sem_w):
        wid = lax.axis_index("s") * _NC + lax.axis_index("c")
        r = wid // w_per_row
        c0 = (wid % w_per_row) * b_per_w
        head = b_per_w - _TAIL
        pltpu.sync_copy(pos_hbm.at[r, pl.ds(c0, b_per_w)], idx_v)
        g_c = pltpu.async_copy(cos_hbm.at[idx_v], cos_v, sem_c)
        g_s = pltpu.async_copy(sin_hbm.at[idx_v.at[pl.ds(0, head)]],
                               sin_v, sem_s)
        g_c.wait()
        w_c = pltpu.async_copy(cos_v, out_slice(cos_out, r, c0, b_per_w),
                               sem_w)
        g_s.wait()
        w_s = pltpu.async_copy(sin_v, out_slice(sin_out, r, c0, head), sem_w)
        w_c.wait()
        g_t = pltpu.async_copy(sin_hbm.at[idx_v.at[pl.ds(head, _TAIL)]],
                               cos_v.at[pl.ds(0, _TAIL)], sem_s)
        g_t.wait()
        w_t = pltpu.async_copy(cos_v.at[pl.ds(0, _TAIL)],
                               out_slice(sin_out, r, c0 + head, _TAIL),
                               sem_w)
        w_s.wait()
        w_t.wait()

    return sc_gather


def kernel(x, position_ids, cos_cached, sin_cached):
    b, s = position_ids.shape
    cos4, sin4 = _build_sc_gather(b, s)(
        position_ids.astype(jnp.int32),
        cos_cached.astype(jnp.float32), sin_cached.astype(jnp.float32))
    return cos4.astype(x.dtype), sin4.astype(x.dtype)
